# baseline (speedup 1.0000x reference)
import functools

import jax
import jax.numpy as jnp
from jax import lax
from jax.experimental import pallas as pl
from jax.experimental.pallas import tpu as pltpu
from jax.experimental.pallas import tpu_sc as plsc

B = 16
TOTAL = 32768
CLS_ID = 101
SEP_ID = 102
PADLEN = 4098
MAXTOK = PADLEN - 2

H0 = 2056
H1 = PADLEN - H0
HALF = 2064
NVEC = HALF // 16
STAGE = HALF + 8
PAD = 8
BUF = STAGE + PAD + 16

_mesh = plsc.VectorSubcoreMesh(core_axis_name="c", subcore_axis_name="s")


@functools.partial(
    pl.kernel,
    out_type=(jax.ShapeDtypeStruct((B, PADLEN), jnp.int32),
              jax.ShapeDtypeStruct((B, PADLEN), jnp.int32)),
    mesh=_mesh,
    compiler_params=pltpu.CompilerParams(
        use_tc_tiling_on_sc=False,
        disable_bounds_checks=True,
        disable_semaphore_checks=True,
        skip_device_barrier=True,
    ),
    scratch_types=[
        pltpu.VMEM((32,), jnp.int32),
        pltpu.VMEM((BUF,), jnp.int32),
        pltpu.VMEM((HALF,), jnp.int32),
        pltpu.VMEM((HALF,), jnp.int32),
    ],
)
def _pad_rows(cu_hbm, flat_hbm, out_hbm, tid_hbm, cu_v, stage_v, row_v, zero_v):
    h = lax.axis_index("c")
    r = lax.axis_index("s")

    pltpu.sync_copy(cu_hbm, cu_v.at[pl.ds(0, B + 1)])
    start = cu_v[pl.ds(r, 16)][0]
    nxt = cu_v[pl.ds(r + 1, 16)][0]
    clen = jnp.minimum(nxt - start, MAXTOK)

    c0 = h * H0
    src_lo = start + c0 - 1
    abase = jnp.clip((jnp.maximum(src_lo, 0) // 8) * 8, 0, TOTAL - STAGE)
    abase = pl.multiple_of(abase, 8)
    pltpu.sync_copy(flat_hbm.at[pl.ds(abase, STAGE)], stage_v.at[pl.ds(PAD, STAGE)])

    shift = src_lo - abase + PAD
    sep_col = clen + 1
    lane = lax.iota(jnp.int32, 16)
    zero16 = jnp.zeros((16,), jnp.int32)

    def body(j, carry):
        l = j * 16 + lane
        col = c0 + l
        base = jnp.clip(shift + j * 16, 0, BUF - 16)
        tok = stage_v[pl.ds(base, 16)]
        val = jnp.where(col == 0, jnp.int32(CLS_ID),
              jnp.where(col == sep_col, jnp.int32(SEP_ID),
              jnp.where(col <= clen, tok, jnp.int32(0))))
        row_v[pl.ds(j * 16, 16)] = val
        zero_v[pl.ds(j * 16, 16)] = zero16
        return carry

    lax.fori_loop(0, NVEC, body, 0)

    @pl.when(h == 0)
    def _():
        pltpu.sync_copy(row_v.at[pl.ds(0, H0)], out_hbm.at[r, pl.ds(0, H0)])
        pltpu.sync_copy(zero_v.at[pl.ds(0, H0)], tid_hbm.at[r, pl.ds(0, H0)])

    @pl.when(h == 1)
    def _():
        pltpu.sync_copy(row_v.at[pl.ds(0, H1)], out_hbm.at[r, pl.ds(H0, H1)])
        pltpu.sync_copy(zero_v.at[pl.ds(0, H1)], tid_hbm.at[r, pl.ds(H0, H1)])


def kernel(flat_ids, cu_seqlens):
    return _pad_rows(cu_seqlens, flat_ids)


# speedup vs baseline: 1.0075x; 1.0075x over previous
import functools

import jax
import jax.numpy as jnp
from jax import lax
from jax.experimental import pallas as pl
from jax.experimental.pallas import tpu as pltpu
from jax.experimental.pallas import tpu_sc as plsc

B = 16
TOTAL = 32768
CLS_ID = 101
SEP_ID = 102
PADLEN = 4098
MAXTOK = PADLEN - 2

H0 = 2056
H1 = PADLEN - H0
HALF = 2064
NVEC = HALF // 16
STAGE = HALF + 8
PAD = 8
BUF = STAGE + PAD + 16

_mesh = plsc.VectorSubcoreMesh(core_axis_name="c", subcore_axis_name="s")


@functools.partial(
    pl.kernel,
    out_type=(jax.ShapeDtypeStruct((B, PADLEN), jnp.int32),
              jax.ShapeDtypeStruct((B, PADLEN), jnp.int32)),
    mesh=_mesh,
    compiler_params=pltpu.CompilerParams(
        use_tc_tiling_on_sc=False,
        disable_bounds_checks=True,
        disable_semaphore_checks=True,
        skip_device_barrier=True,
    ),
    scratch_types=[
        pltpu.VMEM((32,), jnp.int32),
        pltpu.VMEM((BUF,), jnp.int32),
        pltpu.VMEM((HALF,), jnp.int32),
        pltpu.VMEM((HALF,), jnp.int32),
    ],
)
def _pad_rows(cu_hbm, flat_hbm, out_hbm, tid_hbm, cu_v, stage_v, row_v, zero_v):
    h = lax.axis_index("c")
    r = lax.axis_index("s")

    pltpu.sync_copy(cu_hbm, cu_v.at[pl.ds(0, B + 1)])
    start = cu_v[pl.ds(r, 16)][0]
    nxt = cu_v[pl.ds(r + 1, 16)][0]
    clen = jnp.minimum(nxt - start, MAXTOK)

    c0 = h * H0
    src_lo = start + c0 - 1
    abase = jnp.clip((jnp.maximum(src_lo, 0) // 8) * 8, 0, TOTAL - STAGE)
    abase = pl.multiple_of(abase, 8)
    pltpu.sync_copy(flat_hbm.at[pl.ds(abase, STAGE)], stage_v.at[pl.ds(PAD, STAGE)])

    shift = src_lo - abase + PAD
    sep_col = clen + 1
    lane = lax.iota(jnp.int32, 16)
    zero16 = jnp.zeros((16,), jnp.int32)

    def body(j, carry):
        col = c0 + j * 16 + lane
        base = jnp.clip(shift + j * 16, 0, BUF - 16)
        tok = stage_v[pl.ds(base, 16)]
        row_v[pl.ds(j * 16, 16)] = jnp.where(col <= clen, tok, jnp.int32(0))
        zero_v[pl.ds(j * 16, 16)] = zero16
        return carry

    lax.fori_loop(0, NVEC, body, 0)

    @pl.when(h == 0)
    def _():
        v0 = row_v[pl.ds(0, 16)]
        row_v[pl.ds(0, 16)] = jnp.where(lane == 0, jnp.int32(CLS_ID), v0)

    sep_l = sep_col - c0
    @pl.when((sep_l >= 0) & (sep_l < HALF))
    def _():
        jb = (sep_l // 16) * 16
        v = row_v[pl.ds(jb, 16)]
        row_v[pl.ds(jb, 16)] = jnp.where(lane == sep_l - jb, jnp.int32(SEP_ID), v)

    @pl.when(h == 0)
    def _():
        pltpu.sync_copy(row_v.at[pl.ds(0, H0)], out_hbm.at[r, pl.ds(0, H0)])
        pltpu.sync_copy(zero_v.at[pl.ds(0, H0)], tid_hbm.at[r, pl.ds(0, H0)])

    @pl.when(h == 1)
    def _():
        pltpu.sync_copy(row_v.at[pl.ds(0, H1)], out_hbm.at[r, pl.ds(H0, H1)])
        pltpu.sync_copy(zero_v.at[pl.ds(0, H1)], tid_hbm.at[r, pl.ds(H0, H1)])


def kernel(flat_ids, cu_seqlens):
    return _pad_rows(cu_seqlens, flat_ids)


# speedup vs baseline: 1.0377x; 1.0299x over previous
import functools

import jax
import jax.numpy as jnp
from jax import lax
from jax.experimental import pallas as pl
from jax.experimental.pallas import tpu as pltpu
from jax.experimental.pallas import tpu_sc as plsc

B = 16
TOTAL = 32768
CLS_ID = 101
SEP_ID = 102
PADLEN = 4098
MAXTOK = PADLEN - 2

CW = 128
FRONT = 8
SW = CW + 16
WBUF = 176

_mesh = plsc.VectorSubcoreMesh(core_axis_name="c", subcore_axis_name="s")


@functools.partial(
    pl.kernel,
    out_type=jax.ShapeDtypeStruct((B, PADLEN), jnp.int32),
    mesh=_mesh,
    scratch_types=[
        pltpu.VMEM((32,), jnp.int32),
        pltpu.VMEM((B * WBUF,), jnp.int32),
        pltpu.VMEM((B, CW), jnp.int32),
        pltpu.SemaphoreType.DMA,
    ],
)
def _pad_tiles(cu_hbm, flat_hbm, out_hbm, cu_v, sbuf, tv, sem):
    c = lax.axis_index("c")
    s = lax.axis_index("s")
    w = s * 2 + c
    cg0 = pl.multiple_of(w * CW, CW)

    pltpu.sync_copy(cu_hbm, cu_v.at[pl.ds(0, B + 1)])
    lane = lax.iota(jnp.int32, 16)

    def window(i):
        start_i = cu_v[pl.ds(i, 16)][0]
        src_lo = start_i + cg0 - 1
        abase = jnp.clip((jnp.maximum(src_lo, 0) // 8) * 8, 0, TOTAL - SW)
        return src_lo, pl.multiple_of(abase, 8)

    def fire(i, carry):
        _, abase = window(i)
        dst = pl.multiple_of(i * WBUF + FRONT, 8)
        pltpu.async_copy(flat_hbm.at[pl.ds(abase, SW)],
                         sbuf.at[pl.ds(dst, SW)], sem)
        return carry

    lax.fori_loop(0, B, fire, 0)

    def drain(i, carry):
        _, abase = window(i)
        dst = pl.multiple_of(i * WBUF + FRONT, 8)
        pltpu.make_async_copy(flat_hbm.at[pl.ds(abase, SW)],
                              sbuf.at[pl.ds(dst, SW)], sem).wait()
        return carry

    lax.fori_loop(0, B, drain, 0)

    def row(i, carry):
        start_i = cu_v[pl.ds(i, 16)][0]
        nxt_i = cu_v[pl.ds(i + 1, 16)][0]
        clen_i = jnp.minimum(nxt_i - start_i, MAXTOK)
        src_lo, abase = window(i)
        shift = src_lo - abase + FRONT

        def vec(j, carry2):
            col = cg0 + j * 16 + lane
            base = i * WBUF + jnp.clip(shift + j * 16, 0, WBUF - 16)
            tok = sbuf[pl.ds(base, 16)]
            tv[i, pl.ds(j * 16, 16)] = jnp.where(col <= clen_i, tok, jnp.int32(0))
            return carry2

        lax.fori_loop(0, CW // 16, vec, 0)

        sep_l = clen_i + 1 - cg0
        @pl.when((sep_l >= 0) & (sep_l < CW))
        def _():
            jb = (sep_l // 16) * 16
            v = tv[i, pl.ds(jb, 16)]
            tv[i, pl.ds(jb, 16)] = jnp.where(lane == sep_l - jb,
                                             jnp.int32(SEP_ID), v)

        @pl.when(w == 0)
        def _():
            v = tv[i, pl.ds(0, 16)]
            tv[i, pl.ds(0, 16)] = jnp.where(lane == 0, jnp.int32(CLS_ID), v)

        return carry

    lax.fori_loop(0, B, row, 0)

    pltpu.sync_copy(tv.at[pl.ds(0, 8)], out_hbm.at[pl.ds(0, 8), pl.ds(cg0, CW)])
    pltpu.sync_copy(tv.at[pl.ds(8, 8)], out_hbm.at[pl.ds(8, 8), pl.ds(cg0, CW)])


def kernel(flat_ids, cu_seqlens):
    main = _pad_tiles(cu_seqlens, flat_ids)
    starts = cu_seqlens[:B]
    clens = jnp.minimum(cu_seqlens[1:] - cu_seqlens[:-1], MAXTOK)
    tok = flat_ids[jnp.clip(starts + MAXTOK - 1, 0, TOTAL - 1)]
    c0 = jnp.where(clens == MAXTOK, tok,
                   jnp.where(clens == MAXTOK - 1, SEP_ID, 0)).astype(jnp.int32)
    c1 = jnp.where(clens == MAXTOK, SEP_ID, 0).astype(jnp.int32)
    tail = jnp.stack([c0, c1], axis=1)
    padded = lax.dynamic_update_slice(main, tail, (0, MAXTOK))
    type_ids = jnp.zeros_like(padded)
    return padded, type_ids
